# baseline (speedup 1.0000x reference)
import functools

import jax
import jax.numpy as jnp
from jax import lax
from jax.experimental import pallas as pl
from jax.experimental.pallas import tpu as pltpu
from jax.experimental.pallas import tpu_sc as plsc

N = 10000
D = 128
DH = 64
NPAD = 10008
NC = 2
NS = 16
NW = NC * NS
CH = 128
E_PAD = 327680
NROWS = E_PAD // CH
CPT = NROWS // NS
CPT_D = NROWS // NW
RPT = 632
DEG_W = 8

_mesh = plsc.VectorSubcoreMesh(core_axis_name="c", subcore_axis_name="s")
_sc_params = pltpu.CompilerParams(use_tc_tiling_on_sc=False)


@functools.partial(
    pl.kernel,
    out_type=jax.ShapeDtypeStruct((NC, NPAD, DEG_W), jnp.float32),
    mesh=_mesh,
    compiler_params=_sc_params,
    scratch_types=[
        pltpu.VMEM((CPT_D, CH), jnp.int32),
        pltpu.VMEM((CH, DEG_W), jnp.float32),
        pltpu.VMEM((RPT, DEG_W), jnp.float32),
        pltpu.VMEM_SHARED((NPAD, DEG_W), jnp.float32),
    ],
)
def _sc_degree(dst_hbm, out_hbm, dst_v, ones_v, wb, acc):
    cid = lax.axis_index("c")
    sid = lax.axis_index("s")
    wid = cid * NS + sid

    pltpu.sync_copy(dst_hbm.at[pl.ds(wid * CPT_D, CPT_D)], dst_v)

    def _fill(i, _):
        wb[i, pl.ds(0, DEG_W)] = jnp.zeros((DEG_W,), jnp.float32)
        return 0

    lax.fori_loop(0, RPT, _fill, 0)

    def _fill1(i, _):
        ones_v[i, pl.ds(0, DEG_W)] = jnp.full((DEG_W,), 1.0, jnp.float32)
        return 0

    lax.fori_loop(0, CH, _fill1, 0)

    base = pl.multiple_of(jnp.minimum(sid * RPT, NPAD - RPT), 8)
    pltpu.sync_copy(wb, acc.at[pl.ds(base, RPT)])
    plsc.subcore_barrier()

    def _count(j, _):
        pltpu.sync_copy(ones_v, acc.at[dst_v.at[j]], add=True)
        return 0

    lax.fori_loop(0, CPT_D, _count, 0)
    plsc.subcore_barrier()

    pltpu.sync_copy(acc.at[pl.ds(base, RPT)], wb)
    pltpu.sync_copy(wb, out_hbm.at[cid, pl.ds(base, RPT)])


@functools.partial(
    pl.kernel,
    out_type=jax.ShapeDtypeStruct((NC, NPAD, DH), jnp.float32),
    mesh=_mesh,
    compiler_params=_sc_params,
    scratch_types=[
        pltpu.VMEM((CPT, CH), jnp.int32),
        pltpu.VMEM((CPT, CH), jnp.int32),
        pltpu.VMEM((CH, DH), jnp.float32),
        pltpu.VMEM((RPT, DH), jnp.float32),
        pltpu.VMEM_SHARED((NPAD, DH), jnp.float32),
        pltpu.SemaphoreType.DMA,
    ],
)
def _sc_scatter(z_hbm, src_hbm, dst_hbm, out_hbm,
                src_v, dst_v, buf_a, wb, acc, sem_a):
    cid = lax.axis_index("c")
    sid = lax.axis_index("s")

    pltpu.sync_copy(src_hbm.at[cid, pl.ds(sid * CPT, CPT)], src_v)
    pltpu.sync_copy(dst_hbm.at[pl.ds(sid * CPT, CPT)], dst_v)

    def _zero_row(i, _):
        for j in range(DH // 16):
            wb[i, pl.ds(j * 16, 16)] = jnp.zeros((16,), jnp.float32)
        return 0

    lax.fori_loop(0, RPT, _zero_row, 0)

    base = pl.multiple_of(jnp.minimum(sid * RPT, NPAD - RPT), 8)
    pltpu.sync_copy(wb, acc.at[pl.ds(base, RPT)])
    plsc.subcore_barrier()

    def _one(j, _):
        pltpu.async_copy(z_hbm.at[src_v.at[j]], buf_a, sem_a).wait()
        pltpu.sync_copy(buf_a, acc.at[dst_v.at[j]], add=True)
        return 0

    lax.fori_loop(0, CPT, _one, 0)
    plsc.subcore_barrier()

    pltpu.sync_copy(acc.at[pl.ds(base, RPT)], wb)
    pltpu.sync_copy(wb, out_hbm.at[cid, pl.ds(base, RPT)])


def _tc_degsum(dp_ref, o_ref):
    v = dp_ref[0] + dp_ref[1]
    o_ref[...] = lax.rsqrt(v + 1.0)


def _tc_first(x_ref, w_ref, o_ref):
    x = x_ref[...]
    for h in range(2):
        o_ref[h] = jnp.dot(x, w_ref[h], preferred_element_type=jnp.float32)


def _tc_scale(lin_ref, sb_ref, o_ref):
    sb = sb_ref[...]
    o_ref[0] = sb * lin_ref[0]
    o_ref[1] = sb * lin_ref[1]


def _tc_mid(agg_ref, z_ref, sb_ref, wq_ref, b_ref, o_ref):
    sb = sb_ref[...]
    h0 = jnp.maximum(sb * (agg_ref[0] + z_ref[0]) + b_ref[0], 0.0)
    h1 = jnp.maximum(sb * (agg_ref[1] + z_ref[1]) + b_ref[1], 0.0)
    for h in range(2):
        acc = jnp.dot(h0, wq_ref[0, h], preferred_element_type=jnp.float32)
        acc = acc + jnp.dot(h1, wq_ref[1, h], preferred_element_type=jnp.float32)
        o_ref[h] = sb * acc


def _tc_last(agg_ref, z_ref, sb_ref, b_ref, o0_ref, o1_ref):
    sb = sb_ref[...]
    o0_ref[...] = sb * (agg_ref[0] + z_ref[0]) + b_ref[0]
    o1_ref[...] = sb * (agg_ref[1] + z_ref[1]) + b_ref[1]


_BLK = 1000
_GRID = N // _BLK

_x_spec = pl.BlockSpec((_BLK, D), lambda i: (i, 0))
_sb_spec = pl.BlockSpec((_BLK, DH), lambda i: (i, 0))
_half_spec = pl.BlockSpec((NC, _BLK, DH), lambda i: (0, i, 0))
_whalf_spec = pl.BlockSpec((NC, D, DH), lambda i: (0, 0, 0))
_wq_spec = pl.BlockSpec((NC, NC, DH, DH), lambda i: (0, 0, 0, 0))
_bh_spec = pl.BlockSpec((NC, 1, DH), lambda i: (0, 0, 0))
_zhalf_sds = jax.ShapeDtypeStruct((NC, N, DH), jnp.float32)


def _deg_to_s(degp):
    return pl.pallas_call(
        _tc_degsum,
        in_specs=[pl.BlockSpec((NC, NPAD, DEG_W), lambda: (0, 0, 0))],
        out_specs=pl.BlockSpec((NPAD, DEG_W), lambda: (0, 0)),
        out_shape=jax.ShapeDtypeStruct((NPAD, DEG_W), jnp.float32),
    )(degp)


def _first_stage(x, wh):
    return pl.pallas_call(
        _tc_first,
        grid=(_GRID,),
        in_specs=[_x_spec, _whalf_spec],
        out_specs=_half_spec,
        out_shape=_zhalf_sds,
    )(x, wh)


def _scale_stage(lin, sb):
    return pl.pallas_call(
        _tc_scale,
        grid=(_GRID,),
        in_specs=[_half_spec, _sb_spec],
        out_specs=_half_spec,
        out_shape=_zhalf_sds,
    )(lin, sb)


def _mid_stage(agg, z, sb, wq, bh):
    return pl.pallas_call(
        _tc_mid,
        grid=(_GRID,),
        in_specs=[_half_spec, _half_spec, _sb_spec, _wq_spec, _bh_spec],
        out_specs=_half_spec,
        out_shape=_zhalf_sds,
    )(agg, z, sb, wq, bh)


def _last_stage(agg, z, sb, bh):
    lo, hi = pl.pallas_call(
        _tc_last,
        grid=(_GRID,),
        in_specs=[_half_spec, _half_spec, _sb_spec, _bh_spec],
        out_specs=[_sb_spec, _sb_spec],
        out_shape=[jax.ShapeDtypeStruct((N, DH), jnp.float32)] * 2,
    )(agg, z, sb, bh)
    return jnp.concatenate([lo, hi], axis=1)


def _split_w(W):
    return W.reshape(2, DH, 2, DH).transpose(0, 2, 1, 3)


def kernel(x, edge_index, W1, b1, W2, b2, W3, b3):
    src = edge_index[0]
    dst = edge_index[1]
    e = src.shape[0]
    pad = E_PAD - e
    src_p = jnp.concatenate([src, jnp.zeros((pad,), jnp.int32)])
    dst_p = jnp.concatenate([dst, jnp.full((pad,), N, jnp.int32)])
    src2d = src_p.reshape(NROWS, CH)
    dst2d = dst_p.reshape(NROWS, CH)
    src3 = jnp.stack([src2d, src2d + N])

    w1h = W1.reshape(D, 2, DH).transpose(1, 0, 2)
    wq2 = _split_w(W2)
    wq3 = _split_w(W3)
    b1h = b1.reshape(2, 1, DH)
    b2h = b2.reshape(2, 1, DH)
    b3h = b3.reshape(2, 1, DH)

    degp = _sc_degree(dst2d)
    lin1 = _first_stage(x, w1h)
    s8 = _deg_to_s(degp)
    sb = jnp.broadcast_to(s8[:N, :1], (N, DH))

    z1 = _scale_stage(lin1, sb)
    a1 = _sc_scatter(z1.reshape(NC * N, DH), src3, dst2d)
    z2 = _mid_stage(a1, z1, sb, wq2, b1h)
    a2 = _sc_scatter(z2.reshape(NC * N, DH), src3, dst2d)
    z3 = _mid_stage(a2, z2, sb, wq3, b2h)
    a3 = _sc_scatter(z3.reshape(NC * N, DH), src3, dst2d)
    return _last_stage(a3, z3, sb, b3h)


# speedup vs baseline: 1.1115x; 1.1115x over previous
import functools

import jax
import jax.numpy as jnp
from jax import lax
from jax.experimental import pallas as pl
from jax.experimental.pallas import tpu as pltpu
from jax.experimental.pallas import tpu_sc as plsc

N = 10000
D = 128
DH = 64
NPAD = 10008
NC = 2
NS = 16
NW = NC * NS
CH = 128
E_PAD = 327680
NROWS = E_PAD // CH
CPT = NROWS // NS
CPT_D = NROWS // NW
RPT = 632
DEG_W = 8

_mesh = plsc.VectorSubcoreMesh(core_axis_name="c", subcore_axis_name="s")
_sc_params = pltpu.CompilerParams(use_tc_tiling_on_sc=False)


@functools.partial(
    pl.kernel,
    out_type=jax.ShapeDtypeStruct((NC, NPAD, DEG_W), jnp.float32),
    mesh=_mesh,
    compiler_params=_sc_params,
    scratch_types=[
        pltpu.VMEM((CPT_D, CH), jnp.int32),
        pltpu.VMEM((CH, DEG_W), jnp.float32),
        pltpu.VMEM((RPT, DEG_W), jnp.float32),
        pltpu.VMEM_SHARED((NPAD, DEG_W), jnp.float32),
    ],
)
def _sc_degree(dst_hbm, out_hbm, dst_v, ones_v, wb, acc):
    cid = lax.axis_index("c")
    sid = lax.axis_index("s")
    wid = cid * NS + sid

    pltpu.sync_copy(dst_hbm.at[pl.ds(wid * CPT_D, CPT_D)], dst_v)

    def _fill(i, _):
        wb[i, pl.ds(0, DEG_W)] = jnp.zeros((DEG_W,), jnp.float32)
        return 0

    lax.fori_loop(0, RPT, _fill, 0)

    def _fill1(i, _):
        ones_v[i, pl.ds(0, DEG_W)] = jnp.full((DEG_W,), 1.0, jnp.float32)
        return 0

    lax.fori_loop(0, CH, _fill1, 0)

    base = pl.multiple_of(jnp.minimum(sid * RPT, NPAD - RPT), 8)
    pltpu.sync_copy(wb, acc.at[pl.ds(base, RPT)])
    plsc.subcore_barrier()

    def _count(j, _):
        pltpu.sync_copy(ones_v, acc.at[dst_v.at[j]], add=True)
        return 0

    lax.fori_loop(0, CPT_D, _count, 0)
    plsc.subcore_barrier()

    pltpu.sync_copy(acc.at[pl.ds(base, RPT)], wb)
    pltpu.sync_copy(wb, out_hbm.at[cid, pl.ds(base, RPT)])


@functools.partial(
    pl.kernel,
    out_type=jax.ShapeDtypeStruct((NC, NPAD, DH), jnp.float32),
    mesh=_mesh,
    compiler_params=_sc_params,
    scratch_types=[
        pltpu.VMEM((CPT, CH), jnp.int32),
        pltpu.VMEM((CPT, CH), jnp.int32),
        pltpu.VMEM((CH, DH), jnp.float32),
        pltpu.VMEM((RPT, DH), jnp.float32),
        pltpu.VMEM_SHARED((NPAD, DH), jnp.float32),
        pltpu.SemaphoreType.DMA,
    ],
)
def _sc_scatter(z_hbm, src_hbm, dst_hbm, out_hbm,
                src_v, dst_v, buf_a, wb, acc, sem_a):
    cid = lax.axis_index("c")
    sid = lax.axis_index("s")

    pltpu.sync_copy(src_hbm.at[cid, pl.ds(sid * CPT, CPT)], src_v)
    pltpu.sync_copy(dst_hbm.at[pl.ds(sid * CPT, CPT)], dst_v)

    def _zero_row(i, _):
        for j in range(DH // 16):
            wb[i, pl.ds(j * 16, 16)] = jnp.zeros((16,), jnp.float32)
        return 0

    lax.fori_loop(0, RPT, _zero_row, 0)

    base = pl.multiple_of(jnp.minimum(sid * RPT, NPAD - RPT), 8)
    pltpu.sync_copy(wb, acc.at[pl.ds(base, RPT)])
    plsc.subcore_barrier()

    def _one(j, _):
        pltpu.async_copy(z_hbm.at[src_v.at[j]], buf_a, sem_a).wait()
        pltpu.sync_copy(buf_a, acc.at[dst_v.at[j]], add=True)
        return 0

    lax.fori_loop(0, CPT, _one, 0)
    plsc.subcore_barrier()

    pltpu.sync_copy(acc.at[pl.ds(base, RPT)], wb)
    pltpu.sync_copy(wb, out_hbm.at[cid, pl.ds(base, RPT)])


def _tc_degsum(dp_ref, o_ref):
    v = dp_ref[0] + dp_ref[1]
    o_ref[...] = lax.rsqrt(v + 1.0)


def _tc_first(x_ref, sb_ref, w_ref, o_ref):
    sb = sb_ref[...]
    x = x_ref[...]
    for h in range(2):
        o_ref[h] = sb * jnp.dot(x, w_ref[h], preferred_element_type=jnp.float32)


def _tc_mid(agg_ref, z_ref, sb_ref, wq_ref, b_ref, o_ref):
    sb = sb_ref[...]
    h0 = jnp.maximum(sb * (agg_ref[0] + z_ref[0]) + b_ref[0], 0.0)
    h1 = jnp.maximum(sb * (agg_ref[1] + z_ref[1]) + b_ref[1], 0.0)
    for h in range(2):
        acc = jnp.dot(h0, wq_ref[0, h], preferred_element_type=jnp.float32)
        acc = acc + jnp.dot(h1, wq_ref[1, h], preferred_element_type=jnp.float32)
        o_ref[h] = sb * acc


def _tc_last(agg_ref, z_ref, sb_ref, b_ref, o0_ref, o1_ref):
    sb = sb_ref[...]
    o0_ref[...] = sb * (agg_ref[0] + z_ref[0]) + b_ref[0]
    o1_ref[...] = sb * (agg_ref[1] + z_ref[1]) + b_ref[1]


_BLK = 1000
_GRID = N // _BLK

_x_spec = pl.BlockSpec((_BLK, D), lambda i: (i, 0))
_sb_spec = pl.BlockSpec((_BLK, DH), lambda i: (i, 0))
_half_spec = pl.BlockSpec((NC, _BLK, DH), lambda i: (0, i, 0))
_whalf_spec = pl.BlockSpec((NC, D, DH), lambda i: (0, 0, 0))
_wq_spec = pl.BlockSpec((NC, NC, DH, DH), lambda i: (0, 0, 0, 0))
_bh_spec = pl.BlockSpec((NC, 1, DH), lambda i: (0, 0, 0))
_zhalf_sds = jax.ShapeDtypeStruct((NC, N, DH), jnp.float32)


def _deg_to_s(degp):
    return pl.pallas_call(
        _tc_degsum,
        in_specs=[pl.BlockSpec((NC, NPAD, DEG_W), lambda: (0, 0, 0))],
        out_specs=pl.BlockSpec((NPAD, DEG_W), lambda: (0, 0)),
        out_shape=jax.ShapeDtypeStruct((NPAD, DEG_W), jnp.float32),
    )(degp)


def _first_stage(x, sb, wh):
    return pl.pallas_call(
        _tc_first,
        grid=(_GRID,),
        in_specs=[_x_spec, _sb_spec, _whalf_spec],
        out_specs=_half_spec,
        out_shape=_zhalf_sds,
    )(x, sb, wh)


def _mid_stage(agg, z, sb, wq, bh):
    return pl.pallas_call(
        _tc_mid,
        grid=(_GRID,),
        in_specs=[_half_spec, _half_spec, _sb_spec, _wq_spec, _bh_spec],
        out_specs=_half_spec,
        out_shape=_zhalf_sds,
    )(agg, z, sb, wq, bh)


def _last_stage(agg, z, sb, bh):
    lo, hi = pl.pallas_call(
        _tc_last,
        grid=(_GRID,),
        in_specs=[_half_spec, _half_spec, _sb_spec, _bh_spec],
        out_specs=[_sb_spec, _sb_spec],
        out_shape=[jax.ShapeDtypeStruct((N, DH), jnp.float32)] * 2,
    )(agg, z, sb, bh)
    return jnp.concatenate([lo, hi], axis=1)


def _split_w(W):
    return W.reshape(2, DH, 2, DH).transpose(0, 2, 1, 3)


def kernel(x, edge_index, W1, b1, W2, b2, W3, b3):
    src = edge_index[0]
    dst = edge_index[1]
    e = src.shape[0]
    pad = E_PAD - e
    src_p = jnp.concatenate([src, jnp.zeros((pad,), jnp.int32)])
    dst_p = jnp.concatenate([dst, jnp.full((pad,), N, jnp.int32)])
    src2d = src_p.reshape(NROWS, CH)
    dst2d = dst_p.reshape(NROWS, CH)
    src3 = jnp.stack([src2d, src2d + N])

    w1h = W1.reshape(D, 2, DH).transpose(1, 0, 2)
    wq2 = _split_w(W2)
    wq3 = _split_w(W3)
    b1h = b1.reshape(2, 1, DH)
    b2h = b2.reshape(2, 1, DH)
    b3h = b3.reshape(2, 1, DH)

    degp = _sc_degree(dst2d)
    s8 = _deg_to_s(degp)
    sb = jnp.broadcast_to(s8[:N, :1], (N, DH))

    z1 = _first_stage(x, sb, w1h)
    a1 = _sc_scatter(z1.reshape(NC * N, DH), src3, dst2d)
    z2 = _mid_stage(a1, z1, sb, wq2, b1h)
    a2 = _sc_scatter(z2.reshape(NC * N, DH), src3, dst2d)
    z3 = _mid_stage(a2, z2, sb, wq3, b2h)
    a3 = _sc_scatter(z3.reshape(NC * N, DH), src3, dst2d)
    return _last_stage(a3, z3, sb, b3h)
